# baseline (speedup 1.0000x reference)
import jax
import jax.numpy as jnp
from jax.experimental import pallas as pl
from jax.experimental.pallas import tpu as pltpu

_K = 16
_DIMS = [34, 64, 64, 64, 64, 128, 128, 128, 128, 256, 256, 256, 256]
_B = 128
_P = 128
_N = _B * _P
_J = 16
_NJ = _B // _J
_ROWS = _J * _P
_EPS0 = 1e-5
_EPSL = 1e-5 * float(_K)


def _knn_kernel(pts_ref, h0_ref, as_ref, st_ref):
    i = pl.program_id(0)
    r = jax.lax.broadcasted_iota(jnp.int32, (_P, _P), 0)
    c = jax.lax.broadcasted_iota(jnp.int32, (_P, _P), 1)
    for q in range(_J):
        pq = pts_ref[q]
        x = pq[0:1, :]
        y = pq[1:2, :]
        dx = jnp.transpose(x) - x
        dy = jnp.transpose(y) - y
        d2 = dx * dx + dy * dy
        d2 = jnp.where(r == c, 1e9, d2)
        m_t = jnp.zeros((_P, _P), jnp.float32)
        for _ in range(_K):
            m = jnp.min(d2, axis=0, keepdims=True)
            sel = d2 <= m
            m_t = m_t + sel.astype(jnp.float32)
            d2 = jnp.where(sel, 1e9, d2)
        deg = jnp.sum(m_t, axis=1, keepdims=True)
        c_src = jnp.where(deg > 0, jax.lax.rsqrt(deg), 1.0)
        as_ref[q] = m_t * c_src

    @pl.when(i == 0)
    def _():
        st_ref[...] = jnp.zeros_like(st_ref)

    h0 = h0_ref[...]
    st_ref[0:1, :] += jnp.sum(h0, axis=0, keepdims=True)
    st_ref[1:2, :] += jnp.sum(h0 * h0, axis=0, keepdims=True)


def _make_group_kernel(n_layers, dim, out_dim, first_group, has_resid0,
                       with_final):

    def kfn(*refs):
        it = iter(refs)
        a_ref = next(it)
        st_ref = next(it)
        gbs_ref = next(it)
        hp_ref = next(it) if has_resid0 else None
        as_ref = next(it)
        ws_ref = next(it)
        if with_final:
            mw0_ref, mb0_ref = next(it), next(it)
            mw1_ref, mb1_ref = next(it), next(it)
            mw2_ref, mb2_ref = next(it), next(it)
            out_ref = next(it)
        else:
            agg_ref = next(it)
            sto_ref = next(it)
            ho_ref = next(it)
        s_agg = next(it)
        s_h = next(it)
        s_st = next(it)

        i = pl.program_id(0)
        l = i // _NJ
        b = i % _NJ
        row0 = b * _ROWS

        @pl.when(jnp.logical_and(b == 0, l < n_layers))
        def _():
            s_st[pl.ds(l, 1)] = jnp.zeros((1, 8, dim), jnp.float32)

        lm1 = jnp.maximum(l - 1, 0)
        st_prev = jnp.where(l == 0, st_ref[...],
                            s_st[pl.ds(lm1, 1)].reshape(8, dim))
        a = jnp.where(l == 0, a_ref[...], s_agg[pl.ds(row0, _ROWS), :])
        mean = st_prev[0:1, :] / _N
        if first_group:
            eps = jnp.where(l == 0, _EPS0, _EPSL)
        else:
            eps = _EPSL
        var = st_prev[1:2, :] / _N - mean * mean
        gbl = gbs_ref[0]
        scale = gbl[0:1, :] * jax.lax.rsqrt(var + eps)
        shift = gbl[1:2, :] - mean * scale
        h = a * scale + shift
        if first_group:
            h = jnp.where(l == 0, h, jnp.maximum(h, 0.0))
        else:
            h = jnp.maximum(h, 0.0)
        if has_resid0:
            h = h + jnp.where(l == 0, hp_ref[...], 0.0)
        h = h + jnp.where(l >= 2, s_h[pl.ds(row0, _ROWS), :], 0.0)
        s_h[pl.ds(row0, _ROWS), :] = h

        @pl.when(l < n_layers)
        def _():
            y = jnp.dot(h, ws_ref[0], preferred_element_type=jnp.float32)
            y3 = y.reshape(_J, _P, dim)
            agg3 = jax.lax.dot_general(
                as_ref[...], y3, (((1,), (1,)), ((0,), (0,))),
                preferred_element_type=jnp.float32,
                precision=jax.lax.Precision.HIGHEST)
            agg = agg3.reshape(_ROWS, dim)
            s_st[pl.ds(l, 1), 0:1, :] += jnp.sum(agg, axis=0,
                                                 keepdims=True)[None]
            s_st[pl.ds(l, 1), 1:2, :] += jnp.sum(agg * agg, axis=0,
                                                 keepdims=True)[None]
            if with_final:
                s_agg[pl.ds(row0, _ROWS), :] = agg
            else:
                @pl.when(l < n_layers - 1)
                def _():
                    s_agg[pl.ds(row0, _ROWS), :] = agg

                @pl.when(l == n_layers - 1)
                def _():
                    pad = jnp.zeros((_ROWS, out_dim - dim), jnp.float32)
                    agg_ref[...] = jnp.concatenate([agg, pad], axis=1)
                    ho_ref[...] = jnp.concatenate([h, pad], axis=1)

                @pl.when(i == n_layers * _NJ - 1)
                def _():
                    spad = jnp.zeros((8, out_dim - dim), jnp.float32)
                    sto_ref[...] = jnp.concatenate(
                        [s_st[n_layers - 1], spad], axis=1)

        if with_final:
            @pl.when(l == n_layers)
            def _():
                hg = jnp.mean(h.reshape(_J, _P, dim), axis=1)
                z = jnp.dot(hg, mw0_ref[...],
                            preferred_element_type=jnp.float32)
                z = jnp.maximum(z + mb0_ref[0:1, :], 0.0)
                z = jnp.dot(z, mw1_ref[...],
                            preferred_element_type=jnp.float32)
                z = jnp.maximum(z + mb1_ref[0:1, :], 0.0)
                z = jnp.dot(z, mw2_ref[...],
                            preferred_element_type=jnp.float32)
                out_ref[...] = z + mb2_ref[0:1, :]

    return kfn


def _pack_gb(g, b, dim):
    gb = jnp.stack([g, b], axis=0)
    return jnp.pad(gb, ((0, 6), (0, dim - g.shape[0])))


def _pad_w(w, dim):
    return jnp.pad(w, ((0, dim - w.shape[0]), (0, 0)))


def _run_group(a, stats, gbs, hp, adj, ws, dim, out_dim, first_group,
               mlp=None):
    n_layers = 4
    with_final = mlp is not None
    has_resid0 = hp is not None
    n_sweeps = n_layers + (1 if with_final else 0)
    last0 = (n_layers - 1) * _NJ

    def blk_in(i):
        return (jnp.where(i < _NJ, i, 0), 0)

    def blk_out(i):
        return (jnp.where(i >= last0, i % _NJ, 0), 0)

    def blk_out_final(i):
        return (jnp.where(i >= n_layers * _NJ, i % _NJ, 0), 0)

    in_specs = [
        pl.BlockSpec((_ROWS, dim), blk_in),
        pl.BlockSpec((8, dim), lambda i: (0, 0)),
        pl.BlockSpec((1, 8, dim),
                     lambda i: (jnp.minimum(i // _NJ, n_sweeps - 1), 0, 0)),
    ]
    operands = [a, stats, gbs]
    if has_resid0:
        in_specs.append(pl.BlockSpec((_ROWS, dim), blk_in))
        operands.append(hp)
    in_specs += [
        pl.BlockSpec((_J, _P, _P), lambda i: (i % _NJ, 0, 0)),
        pl.BlockSpec((1, dim, dim),
                     lambda i: (jnp.minimum(i // _NJ, n_layers - 1), 0, 0)),
    ]
    operands += [adj, ws]
    if with_final:
        in_specs += [
            pl.BlockSpec((256, 128), lambda i: (0, 0)),
            pl.BlockSpec((1, 128), lambda i: (0, 0)),
            pl.BlockSpec((128, 64), lambda i: (0, 0)),
            pl.BlockSpec((1, 64), lambda i: (0, 0)),
            pl.BlockSpec((64, 128), lambda i: (0, 0)),
            pl.BlockSpec((1, 128), lambda i: (0, 0)),
        ]
        operands += list(mlp)
        out_specs = pl.BlockSpec((_J, 128), blk_out_final)
        out_shape = jax.ShapeDtypeStruct((_B, 128), jnp.float32)
    else:
        out_specs = [
            pl.BlockSpec((_ROWS, out_dim), blk_out),
            pl.BlockSpec((8, out_dim), lambda i: (0, 0)),
            pl.BlockSpec((_ROWS, out_dim), blk_out),
        ]
        out_shape = [
            jax.ShapeDtypeStruct((_N, out_dim), jnp.float32),
            jax.ShapeDtypeStruct((8, out_dim), jnp.float32),
            jax.ShapeDtypeStruct((_N, out_dim), jnp.float32),
        ]

    return pl.pallas_call(
        _make_group_kernel(n_layers, dim, out_dim, first_group, has_resid0,
                           with_final),
        grid=(n_sweeps * _NJ,),
        in_specs=in_specs,
        out_specs=out_specs,
        out_shape=out_shape,
        scratch_shapes=[
            pltpu.VMEM((_N, dim), jnp.float32),
            pltpu.VMEM((_N, dim), jnp.float32),
            pltpu.VMEM((4, 8, dim), jnp.float32),
        ],
    )(*operands)


def kernel(points, features, lorentz_vectors, mask, params):
    del lorentz_vectors, mask
    f32 = jnp.float32

    h0 = jnp.transpose(features, (0, 2, 1)).reshape(_N, _DIMS[0])
    h0 = jnp.pad(h0, ((0, 0), (0, 64 - _DIMS[0])))

    adj, stats = pl.pallas_call(
        _knn_kernel,
        grid=(_NJ,),
        in_specs=[
            pl.BlockSpec((_J, 2, _P), lambda i: (i, 0, 0)),
            pl.BlockSpec((_ROWS, 64), lambda i: (i, 0)),
        ],
        out_specs=[
            pl.BlockSpec((_J, _P, _P), lambda i: (i, 0, 0)),
            pl.BlockSpec((8, 64), lambda i: (0, 0)),
        ],
        out_shape=[
            jax.ShapeDtypeStruct((_B, _P, _P), f32),
            jax.ShapeDtypeStruct((8, 64), f32),
        ],
    )(points, h0)

    gbs = jnp.stack([
        _pack_gb(params['bn_fts_gamma'], params['bn_fts_beta'], 64),
        _pack_gb(params['g0'], params['be0'], 64),
        _pack_gb(params['g1'], params['be1'], 64),
        _pack_gb(params['g2'], params['be2'], 64),
    ])
    ws = jnp.stack([_pad_w(params['W0'], 64),
                    params['W1'], params['W2'], params['W3']])
    a, stats, hp = _run_group(h0, stats, gbs, None, adj, ws, 64, 128, True)

    gbs = jnp.stack([
        _pack_gb(params['g3'], params['be3'], 128),
        _pack_gb(params['g4'], params['be4'], 128),
        _pack_gb(params['g5'], params['be5'], 128),
        _pack_gb(params['g6'], params['be6'], 128),
    ])
    ws = jnp.stack([_pad_w(params['W4'], 128),
                    params['W5'], params['W6'], params['W7']])
    a, stats, hp = _run_group(a, stats, gbs, hp, adj, ws, 128, 256, False)

    gbs = jnp.stack([
        _pack_gb(params['g7'], params['be7'], 256),
        _pack_gb(params['g8'], params['be8'], 256),
        _pack_gb(params['g9'], params['be9'], 256),
        _pack_gb(params['g10'], params['be10'], 256),
        _pack_gb(params['g11'], params['be11'], 256),
    ])
    ws = jnp.stack([_pad_w(params['W8'], 256),
                    params['W9'], params['W10'], params['W11']])
    mlp = (params['MW0'], params['Mb0'].reshape(1, 128),
           params['MW1'], params['Mb1'].reshape(1, 64),
           jnp.pad(params['MW2'], ((0, 0), (0, 128 - 5))),
           jnp.pad(params['Mb2'], (0, 128 - 5)).reshape(1, 128))
    out = _run_group(a, stats, gbs, hp, adj, ws, 256, 256, False, mlp=mlp)
    return out[:, :5]


# speedup vs baseline: 1.0457x; 1.0457x over previous
import jax
import jax.numpy as jnp
from jax.experimental import pallas as pl
from jax.experimental.pallas import tpu as pltpu

_K = 16
_DIMS = [34, 64, 64, 64, 64, 128, 128, 128, 128, 256, 256, 256, 256]
_B = 128
_P = 128
_N = _B * _P
_J = 16
_NJ = _B // _J
_ROWS = _J * _P
_EPS0 = 1e-5
_EPSL = 1e-5 * float(_K)


def _knn_kernel(pts_ref, h0_ref, as_ref, st_ref):
    i = pl.program_id(0)
    p = pts_ref[...]
    x = p[:, 0:1, :]
    y = p[:, 1:2, :]
    dx = jnp.transpose(x, (0, 2, 1)) - x
    dy = jnp.transpose(y, (0, 2, 1)) - y
    d2 = dx * dx + dy * dy
    r = jax.lax.broadcasted_iota(jnp.int32, (_J, _P, _P), 1)
    c = jax.lax.broadcasted_iota(jnp.int32, (_J, _P, _P), 2)
    d2 = jnp.where(r == c, 1e9, d2)
    m_t = jnp.zeros((_J, _P, _P), jnp.float32)
    for _ in range(_K):
        m = jnp.min(d2, axis=1, keepdims=True)
        sel = d2 <= m
        m_t = m_t + sel.astype(jnp.float32)
        d2 = jnp.where(sel, 1e9, d2)
    deg = jnp.sum(m_t, axis=2, keepdims=True)
    c_src = jnp.where(deg > 0, jax.lax.rsqrt(deg), 1.0)
    as_ref[...] = m_t * c_src

    @pl.when(i == 0)
    def _():
        st_ref[...] = jnp.zeros_like(st_ref)

    h0 = h0_ref[...]
    st_ref[0:1, :] += jnp.sum(h0, axis=0, keepdims=True)
    st_ref[1:2, :] += jnp.sum(h0 * h0, axis=0, keepdims=True)


def _make_group_kernel(n_layers, dim, out_dim, first_group, has_resid0,
                       with_final):

    def kfn(*refs):
        it = iter(refs)
        a_ref = next(it)
        st_ref = next(it)
        gbs_ref = next(it)
        hp_ref = next(it) if has_resid0 else None
        as_ref = next(it)
        ws_ref = next(it)
        if with_final:
            mw0_ref, mb0_ref = next(it), next(it)
            mw1_ref, mb1_ref = next(it), next(it)
            mw2_ref, mb2_ref = next(it), next(it)
            out_ref = next(it)
        else:
            agg_ref = next(it)
            sto_ref = next(it)
            ho_ref = next(it)
        s_agg = next(it)
        s_h = next(it)
        s_st = next(it)

        i = pl.program_id(0)
        l = i // _NJ
        b = i % _NJ
        row0 = b * _ROWS

        @pl.when(jnp.logical_and(b == 0, l < n_layers))
        def _():
            s_st[pl.ds(l, 1)] = jnp.zeros((1, 8, dim), jnp.float32)

        lm1 = jnp.maximum(l - 1, 0)
        st_prev = jnp.where(l == 0, st_ref[...],
                            s_st[pl.ds(lm1, 1)].reshape(8, dim))
        a = jnp.where(l == 0, a_ref[...], s_agg[pl.ds(row0, _ROWS), :])
        mean = st_prev[0:1, :] / _N
        if first_group:
            eps = jnp.where(l == 0, _EPS0, _EPSL)
        else:
            eps = _EPSL
        var = st_prev[1:2, :] / _N - mean * mean
        gbl = gbs_ref[0]
        scale = gbl[0:1, :] * jax.lax.rsqrt(var + eps)
        shift = gbl[1:2, :] - mean * scale
        h = a * scale + shift
        if first_group:
            h = jnp.where(l == 0, h, jnp.maximum(h, 0.0))
        else:
            h = jnp.maximum(h, 0.0)
        if has_resid0:
            h = h + jnp.where(l == 0, hp_ref[...], 0.0)
        h = h + jnp.where(l >= 2, s_h[pl.ds(row0, _ROWS), :], 0.0)
        s_h[pl.ds(row0, _ROWS), :] = h

        @pl.when(l < n_layers)
        def _():
            y = jnp.dot(h, ws_ref[0], preferred_element_type=jnp.float32)
            y3 = y.reshape(_J, _P, dim)
            agg3 = jax.lax.dot_general(
                as_ref[...], y3, (((1,), (1,)), ((0,), (0,))),
                preferred_element_type=jnp.float32,
                precision=jax.lax.Precision.HIGHEST)
            agg = agg3.reshape(_ROWS, dim)
            s_st[pl.ds(l, 1), 0:1, :] += jnp.sum(agg, axis=0,
                                                 keepdims=True)[None]
            s_st[pl.ds(l, 1), 1:2, :] += jnp.sum(agg * agg, axis=0,
                                                 keepdims=True)[None]
            if with_final:
                s_agg[pl.ds(row0, _ROWS), :] = agg
            else:
                @pl.when(l < n_layers - 1)
                def _():
                    s_agg[pl.ds(row0, _ROWS), :] = agg

                @pl.when(l == n_layers - 1)
                def _():
                    pad = jnp.zeros((_ROWS, out_dim - dim), jnp.float32)
                    agg_ref[...] = jnp.concatenate([agg, pad], axis=1)
                    ho_ref[...] = jnp.concatenate([h, pad], axis=1)

                @pl.when(i == n_layers * _NJ - 1)
                def _():
                    spad = jnp.zeros((8, out_dim - dim), jnp.float32)
                    sto_ref[...] = jnp.concatenate(
                        [s_st[n_layers - 1], spad], axis=1)

        if with_final:
            @pl.when(l == n_layers)
            def _():
                hg = jnp.mean(h.reshape(_J, _P, dim), axis=1)
                z = jnp.dot(hg, mw0_ref[...],
                            preferred_element_type=jnp.float32)
                z = jnp.maximum(z + mb0_ref[0:1, :], 0.0)
                z = jnp.dot(z, mw1_ref[...],
                            preferred_element_type=jnp.float32)
                z = jnp.maximum(z + mb1_ref[0:1, :], 0.0)
                z = jnp.dot(z, mw2_ref[...],
                            preferred_element_type=jnp.float32)
                out_ref[...] = z + mb2_ref[0:1, :]

    return kfn


def _pack_gb(g, b, dim):
    gb = jnp.stack([g, b], axis=0)
    return jnp.pad(gb, ((0, 6), (0, dim - g.shape[0])))


def _pad_w(w, dim):
    return jnp.pad(w, ((0, dim - w.shape[0]), (0, 0)))


def _run_group(a, stats, gbs, hp, adj, ws, dim, out_dim, first_group,
               mlp=None):
    n_layers = 4
    with_final = mlp is not None
    has_resid0 = hp is not None
    n_sweeps = n_layers + (1 if with_final else 0)
    last0 = (n_layers - 1) * _NJ

    def blk_in(i):
        return (jnp.where(i < _NJ, i, 0), 0)

    def blk_out(i):
        return (jnp.where(i >= last0, i % _NJ, 0), 0)

    def blk_out_final(i):
        return (jnp.where(i >= n_layers * _NJ, i % _NJ, 0), 0)

    in_specs = [
        pl.BlockSpec((_ROWS, dim), blk_in),
        pl.BlockSpec((8, dim), lambda i: (0, 0)),
        pl.BlockSpec((1, 8, dim),
                     lambda i: (jnp.minimum(i // _NJ, n_sweeps - 1), 0, 0)),
    ]
    operands = [a, stats, gbs]
    if has_resid0:
        in_specs.append(pl.BlockSpec((_ROWS, dim), blk_in))
        operands.append(hp)
    in_specs += [
        pl.BlockSpec((_J, _P, _P), lambda i: (i % _NJ, 0, 0)),
        pl.BlockSpec((1, dim, dim),
                     lambda i: (jnp.minimum(i // _NJ, n_layers - 1), 0, 0)),
    ]
    operands += [adj, ws]
    if with_final:
        in_specs += [
            pl.BlockSpec((256, 128), lambda i: (0, 0)),
            pl.BlockSpec((1, 128), lambda i: (0, 0)),
            pl.BlockSpec((128, 64), lambda i: (0, 0)),
            pl.BlockSpec((1, 64), lambda i: (0, 0)),
            pl.BlockSpec((64, 128), lambda i: (0, 0)),
            pl.BlockSpec((1, 128), lambda i: (0, 0)),
        ]
        operands += list(mlp)
        out_specs = pl.BlockSpec((_J, 128), blk_out_final)
        out_shape = jax.ShapeDtypeStruct((_B, 128), jnp.float32)
    else:
        out_specs = [
            pl.BlockSpec((_ROWS, out_dim), blk_out),
            pl.BlockSpec((8, out_dim), lambda i: (0, 0)),
            pl.BlockSpec((_ROWS, out_dim), blk_out),
        ]
        out_shape = [
            jax.ShapeDtypeStruct((_N, out_dim), jnp.float32),
            jax.ShapeDtypeStruct((8, out_dim), jnp.float32),
            jax.ShapeDtypeStruct((_N, out_dim), jnp.float32),
        ]

    return pl.pallas_call(
        _make_group_kernel(n_layers, dim, out_dim, first_group, has_resid0,
                           with_final),
        grid=(n_sweeps * _NJ,),
        in_specs=in_specs,
        out_specs=out_specs,
        out_shape=out_shape,
        scratch_shapes=[
            pltpu.VMEM((_N, dim), jnp.float32),
            pltpu.VMEM((_N, dim), jnp.float32),
            pltpu.VMEM((4, 8, dim), jnp.float32),
        ],
    )(*operands)


def kernel(points, features, lorentz_vectors, mask, params):
    del lorentz_vectors, mask
    f32 = jnp.float32

    h0 = jnp.transpose(features, (0, 2, 1)).reshape(_N, _DIMS[0])
    h0 = jnp.pad(h0, ((0, 0), (0, 64 - _DIMS[0])))

    adj, stats = pl.pallas_call(
        _knn_kernel,
        grid=(_NJ,),
        in_specs=[
            pl.BlockSpec((_J, 2, _P), lambda i: (i, 0, 0)),
            pl.BlockSpec((_ROWS, 64), lambda i: (i, 0)),
        ],
        out_specs=[
            pl.BlockSpec((_J, _P, _P), lambda i: (i, 0, 0)),
            pl.BlockSpec((8, 64), lambda i: (0, 0)),
        ],
        out_shape=[
            jax.ShapeDtypeStruct((_B, _P, _P), f32),
            jax.ShapeDtypeStruct((8, 64), f32),
        ],
    )(points, h0)

    gbs = jnp.stack([
        _pack_gb(params['bn_fts_gamma'], params['bn_fts_beta'], 64),
        _pack_gb(params['g0'], params['be0'], 64),
        _pack_gb(params['g1'], params['be1'], 64),
        _pack_gb(params['g2'], params['be2'], 64),
    ])
    ws = jnp.stack([_pad_w(params['W0'], 64),
                    params['W1'], params['W2'], params['W3']])
    a, stats, hp = _run_group(h0, stats, gbs, None, adj, ws, 64, 128, True)

    gbs = jnp.stack([
        _pack_gb(params['g3'], params['be3'], 128),
        _pack_gb(params['g4'], params['be4'], 128),
        _pack_gb(params['g5'], params['be5'], 128),
        _pack_gb(params['g6'], params['be6'], 128),
    ])
    ws = jnp.stack([_pad_w(params['W4'], 128),
                    params['W5'], params['W6'], params['W7']])
    a, stats, hp = _run_group(a, stats, gbs, hp, adj, ws, 128, 256, False)

    gbs = jnp.stack([
        _pack_gb(params['g7'], params['be7'], 256),
        _pack_gb(params['g8'], params['be8'], 256),
        _pack_gb(params['g9'], params['be9'], 256),
        _pack_gb(params['g10'], params['be10'], 256),
        _pack_gb(params['g11'], params['be11'], 256),
    ])
    ws = jnp.stack([_pad_w(params['W8'], 256),
                    params['W9'], params['W10'], params['W11']])
    mlp = (params['MW0'], params['Mb0'].reshape(1, 128),
           params['MW1'], params['Mb1'].reshape(1, 64),
           jnp.pad(params['MW2'], ((0, 0), (0, 128 - 5))),
           jnp.pad(params['Mb2'], (0, 128 - 5)).reshape(1, 128))
    out = _run_group(a, stats, gbs, hp, adj, ws, 256, 256, False, mlp=mlp)
    return out[:, :5]
